# Initial kernel scaffold; baseline (speedup 1.0000x reference)
import jax
import jax.numpy as jnp
from jax.experimental import pallas as pl


def kernel(x, edge_index, edge_attr, W_i, W_h, W_o, b_o):
    raise NotImplementedError("write your pallas kernel here")


# speedup vs baseline: 1.5756x; 1.5756x over previous
import functools

import jax
import jax.numpy as jnp
from jax import lax
from jax.experimental import pallas as pl
from jax.experimental.pallas import tpu as pltpu
from jax.experimental.pallas import tpu_sc as plsc

_NC = 2
_NS = 16
_CH = 80
_DEPTH = 6
_D = 128

_MESH = functools.partial(
    plsc.VectorSubcoreMesh,
    core_axis_name="c", subcore_axis_name="s", num_cores=_NC, num_subcores=_NS,
)


def _zero_rows(zb, n_rows):
    z16 = jnp.zeros((16,), jnp.float32)

    def body(i, carry):
        r = i // 8
        k = (i % 8) * 16
        zb[r, pl.ds(k, 16)] = z16
        return carry

    lax.fori_loop(0, n_rows * 8, body, 0)


def _zero_acc(acc, zb, sid, n):
    n_ch = n // _CH

    def body(i, carry):
        base = (sid + i * _NS) * _CH
        pltpu.sync_copy(zb, acc.at[pl.ds(base, _CH)])
        return carry

    lax.fori_loop(0, (n_ch - sid + _NS - 1) // _NS, body, 0)


def _sc_gather_x(x, src):
    n, d = x.shape
    e = src.shape[0]
    per_tile = e // (_NC * _NS)
    n_ch = per_tile // _CH

    @functools.partial(
        pl.kernel,
        out_type=jax.ShapeDtypeStruct((e, d), jnp.float32),
        mesh=_MESH(),
        scratch_types=[
            pltpu.VMEM_SHARED((n, d), jnp.float32),
            pltpu.VMEM((1, _CH), jnp.int32),
            pltpu.VMEM((_CH, d), jnp.float32),
        ],
    )
    def k(x_hbm, src_hbm, out_hbm, xs, idx_v, rows_v):
        cid = lax.axis_index("c")
        sid = lax.axis_index("s")

        @pl.when(sid == 0)
        def _():
            pltpu.sync_copy(x_hbm, xs)

        plsc.subcore_barrier()
        gbase = (cid * _NS + sid) * per_tile

        def body(i, carry):
            base = gbase + i * _CH
            pltpu.sync_copy(src_hbm.at[pl.ds(base, _CH)], idx_v.at[0])
            pltpu.sync_copy(xs.at[idx_v.at[0]], rows_v)
            pltpu.sync_copy(rows_v, out_hbm.at[pl.ds(base, _CH)])
            return carry

        lax.fori_loop(0, n_ch, body, 0)

    return k(x, src)


def _sc_seg_gather(msg, dst, n):
    e, d = msg.shape
    sc_per_tile = e // _NS
    sc_n_ch = sc_per_tile // _CH
    g_per_tile = e // (_NC * _NS)
    g_n_ch = g_per_tile // _CH

    @functools.partial(
        pl.kernel,
        out_type=jax.ShapeDtypeStruct((e, d), jnp.float32),
        mesh=_MESH(),
        scratch_types=[
            pltpu.VMEM_SHARED((n, d), jnp.float32),
            pltpu.VMEM((1, _CH), jnp.int32),
            pltpu.VMEM((_CH, d), jnp.float32),
        ],
    )
    def k(msg_hbm, dst_hbm, out_hbm, acc, idx_v, rows_v):
        cid = lax.axis_index("c")
        sid = lax.axis_index("s")
        _zero_rows(rows_v, _CH)
        _zero_acc(acc, rows_v, sid, n)
        plsc.subcore_barrier()

        sbase = sid * sc_per_tile

        def scat(i, carry):
            base = sbase + i * _CH
            pltpu.sync_copy(dst_hbm.at[pl.ds(base, _CH)], idx_v.at[0])
            pltpu.sync_copy(msg_hbm.at[pl.ds(base, _CH)], rows_v)
            pltpu.sync_copy(rows_v, acc.at[idx_v.at[0]], add=True)
            return carry

        lax.fori_loop(0, sc_n_ch, scat, 0)
        plsc.subcore_barrier()

        gbase = (cid * _NS + sid) * g_per_tile

        def gath(i, carry):
            base = gbase + i * _CH
            pltpu.sync_copy(dst_hbm.at[pl.ds(base, _CH)], idx_v.at[0])
            pltpu.sync_copy(acc.at[idx_v.at[0]], rows_v)
            pltpu.sync_copy(rows_v, out_hbm.at[pl.ds(base, _CH)])
            return carry

        lax.fori_loop(0, g_n_ch, gath, 0)

    return k(msg, dst)


def _sc_seg_partial(msg, dst, n):
    e, d = msg.shape
    per_tile = e // (_NC * _NS)
    n_ch = per_tile // _CH
    n_out_ch = n // _CH

    @functools.partial(
        pl.kernel,
        out_type=jax.ShapeDtypeStruct((_NC, n, d), jnp.float32),
        mesh=_MESH(),
        scratch_types=[
            pltpu.VMEM_SHARED((n, d), jnp.float32),
            pltpu.VMEM((1, _CH), jnp.int32),
            pltpu.VMEM((_CH, d), jnp.float32),
        ],
    )
    def k(msg_hbm, dst_hbm, out_hbm, acc, idx_v, rows_v):
        cid = lax.axis_index("c")
        sid = lax.axis_index("s")
        _zero_rows(rows_v, _CH)
        _zero_acc(acc, rows_v, sid, n)
        plsc.subcore_barrier()

        sbase = (cid * _NS + sid) * per_tile

        def scat(i, carry):
            base = sbase + i * _CH
            pltpu.sync_copy(dst_hbm.at[pl.ds(base, _CH)], idx_v.at[0])
            pltpu.sync_copy(msg_hbm.at[pl.ds(base, _CH)], rows_v)
            pltpu.sync_copy(rows_v, acc.at[idx_v.at[0]], add=True)
            return carry

        lax.fori_loop(0, n_ch, scat, 0)
        plsc.subcore_barrier()

        def wout(i, carry):
            base = (sid + i * _NS) * _CH
            pltpu.sync_copy(acc.at[pl.ds(base, _CH)],
                            out_hbm.at[cid, pl.ds(base, _CH)])
            return carry

        lax.fori_loop(0, (n_out_ch - sid + _NS - 1) // _NS, wout, 0)

    return k(msg, dst)


def _tc_init(gx, ea, wa_t, wb_t):
    e, d = gx.shape
    db = ea.shape[1]
    bh = 2000

    def body(gx_ref, ea_ref, wa_ref, wb_ref, inp_ref, msg_ref):
        acc = jnp.dot(gx_ref[...], wa_ref[...], preferred_element_type=jnp.float32)
        acc = acc + jnp.dot(ea_ref[...], wb_ref[...], preferred_element_type=jnp.float32)
        inp_ref[...] = acc
        msg_ref[...] = jnp.maximum(acc, 0.0)

    return pl.pallas_call(
        body,
        grid=(e // bh,),
        in_specs=[
            pl.BlockSpec((bh, d), lambda i: (i, 0)),
            pl.BlockSpec((bh, db), lambda i: (i, 0)),
            pl.BlockSpec((d, d), lambda i: (0, 0)),
            pl.BlockSpec((db, d), lambda i: (0, 0)),
        ],
        out_specs=[
            pl.BlockSpec((bh, d), lambda i: (i, 0)),
            pl.BlockSpec((bh, d), lambda i: (i, 0)),
        ],
        out_shape=[
            jax.ShapeDtypeStruct((e, d), jnp.float32),
            jax.ShapeDtypeStruct((e, d), jnp.float32),
        ],
    )(gx, ea, wa_t, wb_t)


def _tc_depth(ic, mc, gc, wh_t):
    e2, d2 = ic.shape
    d = d2 // 2
    bc = 800

    def body(ic_ref, mc_ref, gc_ref, w_ref, o_ref):
        dm = gc_ref[...] - mc_ref[...]
        w = w_ref[...]
        icv = ic_ref[...]
        o_ref[:, :d] = jnp.maximum(
            icv[:, :d] + jnp.dot(dm[:, d:], w, preferred_element_type=jnp.float32), 0.0)
        o_ref[:, d:] = jnp.maximum(
            icv[:, d:] + jnp.dot(dm[:, :d], w, preferred_element_type=jnp.float32), 0.0)

    return pl.pallas_call(
        body,
        grid=(e2 // bc,),
        in_specs=[
            pl.BlockSpec((bc, d2), lambda i: (i, 0)),
            pl.BlockSpec((bc, d2), lambda i: (i, 0)),
            pl.BlockSpec((bc, d2), lambda i: (i, 0)),
            pl.BlockSpec((d, d), lambda i: (0, 0)),
        ],
        out_specs=pl.BlockSpec((bc, d2), lambda i: (i, 0)),
        out_shape=jax.ShapeDtypeStruct((e2, d2), jnp.float32),
    )(ic, mc, gc, wh_t)


def _tc_final(x, p0, p1, wa_t, wb_t, bo):
    n, d = x.shape
    bn = 1000

    def body(x_ref, p0_ref, p1_ref, wa_ref, wb_ref, b_ref, o_ref):
        s = p0_ref[...] + p1_ref[...]
        acc = jnp.dot(x_ref[...], wa_ref[...], preferred_element_type=jnp.float32)
        acc = acc + jnp.dot(s, wb_ref[...], preferred_element_type=jnp.float32)
        o_ref[...] = jnp.maximum(acc + b_ref[...], 0.0)

    return pl.pallas_call(
        body,
        grid=(n // bn,),
        in_specs=[
            pl.BlockSpec((bn, d), lambda i: (i, 0)),
            pl.BlockSpec((bn, d), lambda i: (i, 0)),
            pl.BlockSpec((bn, d), lambda i: (i, 0)),
            pl.BlockSpec((d, d), lambda i: (0, 0)),
            pl.BlockSpec((d, d), lambda i: (0, 0)),
            pl.BlockSpec((1, d), lambda i: (0, 0)),
        ],
        out_specs=pl.BlockSpec((bn, d), lambda i: (i, 0)),
        out_shape=jax.ShapeDtypeStruct((n, d), jnp.float32),
    )(x, p0, p1, wa_t, wb_t, bo)


def kernel(x, edge_index, edge_attr, W_i, W_h, W_o, b_o):
    n, da = x.shape
    e, db = edge_attr.shape
    src = edge_index[0].astype(jnp.int32)
    dst = edge_index[1].astype(jnp.int32)

    gx = _sc_gather_x(x, src)
    inp, msg = _tc_init(gx, edge_attr,
                        W_i[:, :da].T, W_i[:, da:].T)
    ic = inp.reshape(e // 2, 2 * _D)
    wh_t = W_h.T
    for _ in range(_DEPTH - 1):
        g = _sc_seg_gather(msg, dst, n)
        msg = _tc_depth(ic, msg.reshape(e // 2, 2 * _D),
                        g.reshape(e // 2, 2 * _D), wh_t).reshape(e, _D)
    partials = _sc_seg_partial(msg, dst, n)
    h = _tc_final(x, partials[0], partials[1],
                  W_o[:, :da].T, W_o[:, da:].T, b_o.reshape(1, _D))
    return h


# speedup vs baseline: 2.4677x; 1.5661x over previous
import functools

import jax
import jax.numpy as jnp
from jax import lax
from jax.experimental import pallas as pl
from jax.experimental.pallas import tpu as pltpu
from jax.experimental.pallas import tpu_sc as plsc

_NC = 2
_NS = 16
_CH = 80
_NB = 4
_DEPTH = 6
_D = 128

_MESH = functools.partial(
    plsc.VectorSubcoreMesh,
    core_axis_name="c", subcore_axis_name="s", num_cores=_NC, num_subcores=_NS,
)


def _zero_rows(zb, n_rows):
    z16 = jnp.zeros((16,), jnp.float32)

    def body(i, carry):
        r = i // 8
        k = (i % 8) * 16
        zb[r, pl.ds(k, 16)] = z16
        return carry

    lax.fori_loop(0, n_rows * 8, body, 0)


def _zero_acc(acc, zb, sid, n):
    n_ch = n // _CH

    def body(i, carry):
        base = (sid + i * _NS) * _CH
        pltpu.sync_copy(zb, acc.at[pl.ds(base, _CH)])
        return carry

    lax.fori_loop(0, (n_ch - sid + _NS - 1) // _NS, body, 0)


def _pipe(n_ch, s0, w0, s1, w1, s2=None, w2=None):
    nb = _NB

    def step(c, b, py):
        w0(c, b)
        if s2 is not None and (not py or c >= nb):
            w2(c - nb, b)
        s1(c, b)
        if not py or c >= 1:
            cp, bp = c - 1, (b - 1) % nb
            w1(cp, bp)
            if s2 is not None:
                s2(cp, bp)
            if py:
                if cp + nb < n_ch:
                    s0(cp + nb, bp)
            else:
                @pl.when(cp + nb < n_ch)
                def _():
                    s0(cp + nb, bp)

    for c in range(min(nb, n_ch)):
        s0(c, c)
    for c in range(min(nb, n_ch)):
        step(c, c, True)
    rest = max(0, n_ch - nb)
    gful = rest // nb

    def body(i, carry):
        base = nb + i * nb
        for b in range(nb):
            step(base + b, b, False)
        return carry

    if gful > 0:
        lax.fori_loop(0, gful, body, 0)
    for c in range(nb + gful * nb, n_ch):
        step(c, c % nb, True)
    cl = n_ch - 1
    w1(cl, cl % nb)
    if s2 is not None:
        s2(cl, cl % nb)
        for c in range(max(0, n_ch - nb), n_ch):
            w2(c, c % nb)


def _sc_gather_x(x, src):
    n, d = x.shape
    e = src.shape[0]
    per_tile = e // (_NC * _NS)
    n_ch = per_tile // _CH

    @functools.partial(
        pl.kernel,
        out_type=jax.ShapeDtypeStruct((e, d), jnp.float32),
        mesh=_MESH(),
        scratch_types=[pltpu.VMEM_SHARED((n, d), jnp.float32)]
          + [pltpu.VMEM((1, _CH), jnp.int32)] * _NB
          + [pltpu.VMEM((_CH, d), jnp.float32)] * _NB
          + [pltpu.SemaphoreType.DMA] * (3 * _NB),
    )
    def k(x_hbm, src_hbm, out_hbm, xs, *scr):
        cid = lax.axis_index("c")
        sid = lax.axis_index("s")
        ixb = scr[:_NB]
        rbs = scr[_NB:2 * _NB]
        li = scr[2 * _NB:3 * _NB]
        lr = scr[3 * _NB:4 * _NB]
        so = scr[4 * _NB:]

        @pl.when(sid == 0)
        def _():
            pltpu.sync_copy(x_hbm, xs)

        plsc.subcore_barrier()
        gbase = (cid * _NS + sid) * per_tile

        def s0(c, b):
            pltpu.async_copy(src_hbm.at[pl.ds(gbase + c * _CH, _CH)], ixb[b].at[0], li[b])

        def w0(c, b):
            pltpu.make_async_copy(
                src_hbm.at[pl.ds(gbase + c * _CH, _CH)], ixb[b].at[0], li[b]).wait()

        def s1(c, b):
            pltpu.async_copy(xs.at[ixb[b].at[0]], rbs[b], lr[b])

        def w1(c, b):
            pltpu.make_async_copy(xs.at[ixb[b].at[0]], rbs[b], lr[b]).wait()

        def s2(c, b):
            pltpu.async_copy(rbs[b], out_hbm.at[pl.ds(gbase + c * _CH, _CH)], so[b])

        def w2(c, b):
            pltpu.make_async_copy(
                rbs[b], out_hbm.at[pl.ds(gbase + c * _CH, _CH)], so[b]).wait()

        _pipe(n_ch, s0, w0, s1, w1, s2, w2)

    return k(x, src)


def _sc_scat_h(msg_h, dst, n, h):
    e2, d = msg_h.shape
    chs = 40
    per_tile = e2 // (_NC * _NS)
    n_ch = per_tile // chs
    n_out_ch = n // _CH

    @functools.partial(
        pl.kernel,
        out_type=jax.ShapeDtypeStruct((_NC, n, d), jnp.float32),
        mesh=_MESH(),
        scratch_types=[pltpu.VMEM_SHARED((n, d), jnp.float32)]
          + [pltpu.VMEM((1, chs), jnp.int32)] * _NB
          + [pltpu.VMEM((_CH, d), jnp.float32)] * _NB
          + [pltpu.SemaphoreType.DMA] * (3 * _NB),
    )
    def k(msg_hbm, dst_hbm, out_hbm, acc, *scr):
        cid = lax.axis_index("c")
        sid = lax.axis_index("s")
        ixb = scr[:_NB]
        rbs = scr[_NB:2 * _NB]
        li = scr[2 * _NB:3 * _NB]
        lr = scr[3 * _NB:4 * _NB]
        so = scr[4 * _NB:]

        _zero_rows(rbs[0], _CH)
        _zero_acc(acc, rbs[0], sid, n)
        plsc.subcore_barrier()

        wid = cid * _NS + sid
        mbase = wid * per_tile
        dbase = h * e2 + mbase

        def s_s0(c, b):
            pltpu.async_copy(dst_hbm.at[pl.ds(dbase + c * chs, chs)], ixb[b].at[0], li[b])
            pltpu.async_copy(msg_hbm.at[pl.ds(mbase + c * chs, chs)],
                             rbs[b].at[pl.ds(0, chs)], lr[b])

        def s_w0(c, b):
            pltpu.make_async_copy(
                dst_hbm.at[pl.ds(dbase + c * chs, chs)], ixb[b].at[0], li[b]).wait()
            pltpu.make_async_copy(
                msg_hbm.at[pl.ds(mbase + c * chs, chs)],
                rbs[b].at[pl.ds(0, chs)], lr[b]).wait()

        def s_s1(c, b):
            pltpu.async_copy(rbs[b].at[pl.ds(0, chs)], acc.at[ixb[b].at[0]],
                             so[b], add=True)

        def s_w1(c, b):
            pltpu.make_async_copy(rbs[b].at[pl.ds(0, chs)],
                                  acc.at[ixb[b].at[0]], so[b]).wait()

        _pipe(n_ch, s_s0, s_w0, s_s1, s_w1)
        plsc.subcore_barrier()

        def wout(i, carry):
            base = (sid + i * _NS) * _CH
            pltpu.sync_copy(acc.at[pl.ds(base, _CH)],
                            out_hbm.at[cid, pl.ds(base, _CH)])
            return carry

        lax.fori_loop(0, (n_out_ch - sid + _NS - 1) // _NS, wout, 0)

    return k(msg_h, dst)


def _tc_sum4(q0, q1):
    _, n, d = q0.shape
    bn = 1000

    def body(a_ref, b_ref, o_ref):
        o_ref[...] = (a_ref[0] + a_ref[1]) + (b_ref[0] + b_ref[1])

    return pl.pallas_call(
        body,
        grid=(n // bn,),
        in_specs=[
            pl.BlockSpec((2, bn, d), lambda i: (0, i, 0)),
            pl.BlockSpec((2, bn, d), lambda i: (0, i, 0)),
        ],
        out_specs=pl.BlockSpec((bn, d), lambda i: (i, 0)),
        out_shape=jax.ShapeDtypeStruct((n, d), jnp.float32),
    )(q0, q1)


def _sc_gath_h(esum, dst, h):
    n, d = esum.shape
    e2 = dst.shape[0] // 2
    per_tile = e2 // (_NC * _NS)
    ch2 = 40
    n_ch = per_tile // ch2

    @functools.partial(
        pl.kernel,
        out_type=jax.ShapeDtypeStruct((e2, d), jnp.float32),
        mesh=_MESH(),
        scratch_types=[pltpu.VMEM_SHARED((n, d), jnp.float32)]
          + [pltpu.VMEM((1, _CH), jnp.int32)] * _NB
          + [pltpu.VMEM((_CH, d), jnp.float32)] * _NB
          + [pltpu.SemaphoreType.DMA] * (3 * _NB),
    )
    def k(es_hbm, dst_hbm, out_hbm, acc, *scr):
        cid = lax.axis_index("c")
        sid = lax.axis_index("s")
        ixb = scr[:_NB]
        rbs = scr[_NB:2 * _NB]
        li = scr[2 * _NB:3 * _NB]
        lr = scr[3 * _NB:4 * _NB]
        so = scr[4 * _NB:]

        @pl.when(sid == 0)
        def _():
            pltpu.sync_copy(es_hbm, acc)

        plsc.subcore_barrier()

        wid = cid * _NS + sid
        gbase = wid * per_tile
        dbase = h * e2 + gbase

        def g_s0(c, b):
            pltpu.async_copy(dst_hbm.at[pl.ds(dbase + c * ch2, ch2)],
                             ixb[b].at[0, pl.ds(0, ch2)], li[b])

        def g_w0(c, b):
            pltpu.make_async_copy(dst_hbm.at[pl.ds(dbase + c * ch2, ch2)],
                                  ixb[b].at[0, pl.ds(0, ch2)], li[b]).wait()

        def g_s1(c, b):
            pltpu.async_copy(acc.at[ixb[b].at[0, pl.ds(0, ch2)]],
                             rbs[b].at[pl.ds(0, ch2)], lr[b])

        def g_w1(c, b):
            pltpu.make_async_copy(acc.at[ixb[b].at[0, pl.ds(0, ch2)]],
                                  rbs[b].at[pl.ds(0, ch2)], lr[b]).wait()

        def g_s2(c, b):
            pltpu.async_copy(rbs[b].at[pl.ds(0, ch2)],
                             out_hbm.at[pl.ds(gbase + c * ch2, ch2)], so[b])

        def g_w2(c, b):
            pltpu.make_async_copy(rbs[b].at[pl.ds(0, ch2)],
                                  out_hbm.at[pl.ds(gbase + c * ch2, ch2)], so[b]).wait()

        _pipe(n_ch, g_s0, g_w0, g_s1, g_w1, g_s2, g_w2)

    return k(esum, dst)


def _tc_init_h(gx, ea, wa_t, wb_t, h):
    e, d = gx.shape
    e2 = e // 2
    db = ea.shape[1]
    bh = 2000
    nblk = e2 // bh

    def body(gx_ref, ea_ref, wa_ref, wb_ref, inp_ref, msg_ref):
        acc = jnp.dot(gx_ref[...], wa_ref[...], preferred_element_type=jnp.float32)
        acc = acc + jnp.dot(ea_ref[...], wb_ref[...], preferred_element_type=jnp.float32)
        inp_ref[...] = acc
        msg_ref[...] = jnp.maximum(acc, 0.0)

    return pl.pallas_call(
        body,
        grid=(nblk,),
        in_specs=[
            pl.BlockSpec((bh, d), lambda i: (i + h * nblk, 0)),
            pl.BlockSpec((bh, db), lambda i: (i + h * nblk, 0)),
            pl.BlockSpec((d, d), lambda i: (0, 0)),
            pl.BlockSpec((db, d), lambda i: (0, 0)),
        ],
        out_specs=[
            pl.BlockSpec((bh, d), lambda i: (i, 0)),
            pl.BlockSpec((bh, d), lambda i: (i, 0)),
        ],
        out_shape=[
            jax.ShapeDtypeStruct((e2, d), jnp.float32),
            jax.ShapeDtypeStruct((e2, d), jnp.float32),
        ],
    )(gx, ea, wa_t, wb_t)


def _tc_depth(ic, mc, gc, wh_t):
    e2, d2 = ic.shape
    d = d2 // 2
    bc = 800

    def body(ic_ref, mc_ref, gc_ref, w_ref, o_ref):
        dm = gc_ref[...] - mc_ref[...]
        w = w_ref[...]
        icv = ic_ref[...]
        o_ref[:, :d] = jnp.maximum(
            icv[:, :d] + jnp.dot(dm[:, d:], w, preferred_element_type=jnp.float32), 0.0)
        o_ref[:, d:] = jnp.maximum(
            icv[:, d:] + jnp.dot(dm[:, :d], w, preferred_element_type=jnp.float32), 0.0)

    return pl.pallas_call(
        body,
        grid=(e2 // bc,),
        in_specs=[
            pl.BlockSpec((bc, d2), lambda i: (i, 0)),
            pl.BlockSpec((bc, d2), lambda i: (i, 0)),
            pl.BlockSpec((bc, d2), lambda i: (i, 0)),
            pl.BlockSpec((d, d), lambda i: (0, 0)),
        ],
        out_specs=pl.BlockSpec((bc, d2), lambda i: (i, 0)),
        out_shape=jax.ShapeDtypeStruct((e2, d2), jnp.float32),
    )(ic, mc, gc, wh_t)


def _tc_final(x, esum, wa_t, wb_t, bo):
    n, d = x.shape
    bn = 1000

    def body(x_ref, s_ref, wa_ref, wb_ref, b_ref, o_ref):
        acc = jnp.dot(x_ref[...], wa_ref[...], preferred_element_type=jnp.float32)
        acc = acc + jnp.dot(s_ref[...], wb_ref[...], preferred_element_type=jnp.float32)
        o_ref[...] = jnp.maximum(acc + b_ref[...], 0.0)

    return pl.pallas_call(
        body,
        grid=(n // bn,),
        in_specs=[
            pl.BlockSpec((bn, d), lambda i: (i, 0)),
            pl.BlockSpec((bn, d), lambda i: (i, 0)),
            pl.BlockSpec((d, d), lambda i: (0, 0)),
            pl.BlockSpec((d, d), lambda i: (0, 0)),
            pl.BlockSpec((1, d), lambda i: (0, 0)),
        ],
        out_specs=pl.BlockSpec((bn, d), lambda i: (i, 0)),
        out_shape=jax.ShapeDtypeStruct((n, d), jnp.float32),
    )(x, esum, wa_t, wb_t, bo)


def kernel(x, edge_index, edge_attr, W_i, W_h, W_o, b_o):
    n, da = x.shape
    e, db = edge_attr.shape
    e2 = e // 2
    src = edge_index[0].astype(jnp.int32)
    dst = edge_index[1].astype(jnp.int32)

    gx = _sc_gather_x(x, src)
    wia_t = W_i[:, :da].T
    wib_t = W_i[:, da:].T
    inp0, msg0 = _tc_init_h(gx, edge_attr, wia_t, wib_t, 0)
    inp1, msg1 = _tc_init_h(gx, edge_attr, wia_t, wib_t, 1)
    ic0 = inp0.reshape(e2 // 2, 2 * _D)
    ic1 = inp1.reshape(e2 // 2, 2 * _D)
    wh_t = W_h.T
    for _ in range(_DEPTH - 1):
        q0 = _sc_scat_h(msg0, dst, n, 0)
        q1 = _sc_scat_h(msg1, dst, n, 1)
        esum = _tc_sum4(q0, q1)
        g0 = _sc_gath_h(esum, dst, 0)
        g1 = _sc_gath_h(esum, dst, 1)
        msg0 = _tc_depth(ic0, msg0.reshape(e2 // 2, 2 * _D),
                         g0.reshape(e2 // 2, 2 * _D), wh_t).reshape(e2, _D)
        msg1 = _tc_depth(ic1, msg1.reshape(e2 // 2, 2 * _D),
                         g1.reshape(e2 // 2, 2 * _D), wh_t).reshape(e2, _D)
    q0 = _sc_scat_h(msg0, dst, n, 0)
    q1 = _sc_scat_h(msg1, dst, n, 1)
    esum = _tc_sum4(q0, q1)
    h = _tc_final(x, esum,
                  W_o[:, :da].T, W_o[:, da:].T, b_o.reshape(1, _D))
    return h
